# baseline (speedup 1.0000x reference)
import jax
import jax.numpy as jnp
from jax import lax
from jax.experimental import pallas as pl
from jax.experimental.pallas import tpu as pltpu
from jax.experimental.pallas import tpu_sc as plsc

HS = 64
HS1 = HS + 1
L = 200
V_FEAT = 100000
V_ITEM = 1000000
NT = 16
MAT_W = 256
PREF0 = 16
SPARE0 = 72


def _fm_body(featT_hbm, itemsT_hbm, user_hbm, bias_hbm, ui_hbm, pref_hbm,
             mat_hbm, res_hbm,
             idx_v, uiv, ubuf, ibuf4, rowstage, rowbuf, ctile_v, sall_v,
             bias_v, res_v, sem, sem_win, sem_out):
    core = lax.axis_index("c")
    wid = lax.axis_index("s")
    iota = lax.iota(jnp.int32, 16)

    pltpu.sync_copy(pref_hbm.at[0], idx_v.at[pl.ds(0, L)])
    c12 = idx_v[pl.ds(192, 16)]
    idx_v[pl.ds(192, 16)] = jnp.where(lax.iota(jnp.int32, 16) < 8, c12, 0)
    pltpu.sync_copy(ui_hbm.at[0], uiv.at[pl.ds(0, 2)])
    pltpu.sync_copy(user_hbm.at[0], ubuf.at[pl.ds(0, HS1)])
    i_item = uiv[pl.ds(0, 16)][1]
    woff = (i_item // 64) * 64
    lane_it = i_item - woff

    acc = jnp.zeros((16,), jnp.float32)
    res_v[pl.ds(0, 16)] = acc

    def extract_ue_ie(d, b):
        ue = jnp.float32(0.0)
        for k in range(5):
            uc = ubuf[pl.ds(16 * k, 16)]
            ue = ue + jnp.sum(jnp.where(iota + (16 * k) == d, uc, 0.0))
        ie = jnp.float32(0.0)
        for k in range(4):
            ic = ibuf4[b, pl.ds(16 * k, 16)]
            ie = ie + jnp.sum(jnp.where(iota + (16 * k) == lane_it, ic, 0.0))
        return ue, ie

    def gather_row(d, b, ue, ie):
        rowbuf[b, pl.ds(0, 16)] = (jnp.where(iota == 14, ue, 0.0)
                                   + jnp.where(iota == 15, ie, 0.0))
        psum = jnp.float32(0.0)
        for c in range(13):
            idxc = idx_v[pl.ds(16 * c, 16)]
            g = plsc.load_gather(rowstage, [idxc])
            valid = iota + (16 * c) < L
            psum = psum + jnp.sum(jnp.where(valid, g, 0.0))
            rowbuf[b, pl.ds(PREF0 + 16 * c, 16)] = g
        return psum, pltpu.async_copy(rowbuf.at[b], mat_hbm.at[d], sem_out)

    @pl.when(core == 0)
    def _():
        win = [pltpu.async_copy(itemsT_hbm.at[wid + NT * s, pl.ds(woff, 64)],
                                ibuf4.at[s], sem_win) for s in range(4)]
        hout_prev = None
        for slot in range(4):
            d = wid + NT * slot
            hstage = pltpu.async_copy(featT_hbm.at[d], rowstage, sem)
            if slot == 0:
                for h in win:
                    h.wait()
            ue, ie = extract_ue_ie(d, slot)
            hstage.wait()
            if hout_prev is not None:
                hout_prev.wait()
            psum, hout_prev = gather_row(d, slot % 2, ue, ie)
            contrib = ue * ie + (ue + ie) * psum
            cur = res_v[pl.ds(0, 16)]
            res_v[pl.ds(0, 16)] = cur + jnp.where(iota == 0, contrib, 0.0)
        hout_prev.wait()
        ctile_v[pl.ds(0, 16)] = res_v[pl.ds(0, 16)]
        zero16 = jnp.zeros((16,), jnp.float32)
        for k in range(1, MAT_W // 16):
            ctile_v[pl.ds(16 * k, 16)] = zero16
        pltpu.sync_copy(ctile_v, mat_hbm.at[wid + SPARE0])

    @pl.when((core == 1) & (wid == 0))
    def _():
        d64 = wid + HS
        pltpu.sync_copy(itemsT_hbm.at[d64, pl.ds(woff, 64)], ibuf4.at[0])
        pltpu.sync_copy(featT_hbm.at[d64], rowstage)
        ue, ie = extract_ue_ie(d64, 0)
        _, hout = gather_row(d64, 0, ue, ie)
        hout.wait()

    plsc.subcore_barrier()

    @pl.when((core == 0) & (wid == 0))
    def _():
        pltpu.sync_copy(bias_hbm, bias_v.at[pl.ds(0, 1)])
        pltpu.sync_copy(mat_hbm.at[pl.ds(SPARE0, NT)], sall_v)
        tot = jnp.zeros((16,), jnp.float32)
        for s in range(NT):
            tot = tot + sall_v[s, pl.ds(0, 16)]
        total = tot[0] + bias_v[pl.ds(0, 16)][0]
        res_v[pl.ds(0, 16)] = jnp.full((16,), total, jnp.float32)
        pltpu.sync_copy(res_v.at[pl.ds(0, 1)], res_hbm)


def kernel(items_emb, feature_emb, user_emb, Bias, ui_pair, feature_index,
           preference_index):
    del feature_index
    mesh = plsc.VectorSubcoreMesh(core_axis_name="c", subcore_axis_name="s",
                                  num_cores=2)
    fn = pl.kernel(
        _fm_body,
        mesh=mesh,
        out_type=(
            jax.ShapeDtypeStruct((SPARE0 + NT, MAT_W), jnp.float32),
            jax.ShapeDtypeStruct((1,), jnp.float32),
        ),
        scratch_types=[
            pltpu.VMEM((208,), jnp.int32),
            pltpu.VMEM((16,), jnp.int32),
            pltpu.VMEM((80,), jnp.float32),
            pltpu.VMEM((4, 64), jnp.float32),
            pltpu.VMEM((V_FEAT,), jnp.float32),
            pltpu.VMEM((2, MAT_W), jnp.float32),
            pltpu.VMEM((MAT_W,), jnp.float32),
            pltpu.VMEM((NT, MAT_W), jnp.float32),
            pltpu.VMEM((16,), jnp.float32),
            pltpu.VMEM((16,), jnp.float32),
            pltpu.SemaphoreType.DMA,
            pltpu.SemaphoreType.DMA,
            pltpu.SemaphoreType.DMA,
        ],
        compiler_params=pltpu.CompilerParams(needs_layout_passes=False),
    )
    matT, res = fn(feature_emb.T, items_emb.T, user_emb, Bias, ui_pair,
                   preference_index)
    result = res.reshape(1, 1)
    mat = matT[:HS1, PREF0 - 2:PREF0 + L].T
    fb = mat[None, :, HS:]
    nz = mat[None, :, :HS]
    return (result, fb, nz)
